# baseline (speedup 1.0000x reference)
import functools

import jax
import jax.numpy as jnp
from jax import lax
from jax.experimental import pallas as pl
from jax.experimental.pallas import tpu as pltpu
from jax.experimental.pallas import tpu_sc as plsc

D_EMB = 1024
D_MODEL = 2048
B_DIM = 4
L_DIM = 4096
N_TOK = B_DIM * L_DIM

NC = 2
NS = 16
NW = NC * NS
TW = N_TOK // NW
C = 16
NCHUNK = TW // C
NBUF = 3


def _proj_body(emb_ref, w_ref, b_ref, out_ref):
    out_ref[...] = (
        jnp.dot(emb_ref[...], w_ref[...], preferred_element_type=jnp.float32)
        + b_ref[...][None, :]
    )


def _compute_proj(emb_table, W, b):
    return pl.pallas_call(
        _proj_body,
        out_shape=jax.ShapeDtypeStruct((2, D_MODEL), jnp.float32),
    )(emb_table, W, b)


@functools.partial(
    pl.kernel,
    out_type=jax.ShapeDtypeStruct((N_TOK, D_MODEL), jnp.float32),
    mesh=plsc.VectorSubcoreMesh(core_axis_name="c", subcore_axis_name="s"),
    compiler_params=pltpu.CompilerParams(needs_layout_passes=False),
    scratch_types=[
        pltpu.VMEM((TW,), jnp.int32),
        pltpu.VMEM((2 * D_MODEL,), jnp.float32),
        pltpu.VMEM((C, D_MODEL), jnp.float32),
        pltpu.VMEM((C, D_MODEL), jnp.float32),
        pltpu.VMEM((C, D_MODEL), jnp.float32),
        pltpu.SemaphoreType.DMA,
        pltpu.SemaphoreType.DMA,
        pltpu.SemaphoreType.DMA,
    ],
)
def _sc_emit(x_hbm, proj_hbm, out_hbm, idx_v, proj_v, b0, b1, b2,
             ws0, ws1, ws2):
    cid = lax.axis_index("c")
    sid = lax.axis_index("s")
    wid = sid * NC + cid
    base = wid * TW
    pltpu.sync_copy(x_hbm.at[pl.ds(base, TW)], idx_v)
    pltpu.sync_copy(proj_hbm, proj_v)

    bufs = (b0, b1, b2)
    wsems = (ws0, ws1, ws2)
    lanes = lax.iota(jnp.int32, 16)

    def build(ci, p):
        xv = idx_v[pl.ds(ci * C, 16)]
        ws = []
        for i in range(C):
            si = jnp.sum(jnp.where(lanes == i, xv, 0))
            ws.append(jnp.full((16,), si, jnp.int32).astype(jnp.float32))

        def col(j, carry):
            o = j * 16
            p0 = proj_v[pl.ds(o, 16)]
            p1 = proj_v[pl.ds(D_MODEL + o, 16)]
            d = p1 - p0
            for i in range(C):
                bufs[p][i, pl.ds(o, 16)] = p0 + ws[i] * d
            return carry

        lax.fori_loop(0, D_MODEL // 16, col, 0, unroll=8)

    def start_write(ci, p):
        pltpu.async_copy(bufs[p], out_hbm.at[pl.ds(base + ci * C, C)], wsems[p])

    def wait_write(ci, p):
        pltpu.make_async_copy(
            bufs[p], out_hbm.at[pl.ds(base + ci * C, C)], wsems[p]
        ).wait()

    build(0, 0)
    start_write(0, 0)
    build(1, 1)
    start_write(1, 1)

    def body(g, carry):
        for bq in range(NBUF):
            ci = 2 + g * NBUF + bq
            p = (2 + bq) % NBUF

            @pl.when(ci >= NBUF)
            def _():
                wait_write(ci - NBUF, p)

            build(ci, p)
            start_write(ci, p)
        return carry

    lax.fori_loop(0, (NCHUNK - 2) // NBUF, body, 0)
    wait_write(NCHUNK - 3, (NCHUNK - 3) % NBUF)
    wait_write(NCHUNK - 2, (NCHUNK - 2) % NBUF)
    wait_write(NCHUNK - 1, (NCHUNK - 1) % NBUF)


def kernel(x, emb_table, W, b):
    proj = _compute_proj(emb_table, W, b)
    xf = x.reshape(-1).astype(jnp.int32)
    out = _sc_emit(xf, proj.reshape(-1))
    return out.reshape(B_DIM, L_DIM, D_MODEL)


# speedup vs baseline: 1.0300x; 1.0300x over previous
import functools

import jax
import jax.numpy as jnp
from jax import lax
from jax.experimental import pallas as pl
from jax.experimental.pallas import tpu as pltpu
from jax.experimental.pallas import tpu_sc as plsc

D_EMB = 1024
D_MODEL = 2048
B_DIM = 4
L_DIM = 4096
N_TOK = B_DIM * L_DIM

NC = 2
NS = 16
NW = NC * NS
TW = N_TOK // NW
C = 16
NCHUNK = TW // C
NBUF = 3


def _tc_body(x_ref, emb_ref, w_ref, b_ref, proj_ref, xf_ref):
    proj_ref[...] = (
        jnp.dot(emb_ref[...], w_ref[...], preferred_element_type=jnp.float32)
        + b_ref[...][None, :]
    )
    xf_ref[...] = x_ref[...].astype(jnp.float32)


def _tc_prep(x, emb_table, W, b):
    return pl.pallas_call(
        _tc_body,
        out_shape=[
            jax.ShapeDtypeStruct((2, D_MODEL), jnp.float32),
            jax.ShapeDtypeStruct((B_DIM, L_DIM), jnp.float32),
        ],
    )(x, emb_table, W, b)


@functools.partial(
    pl.kernel,
    out_type=jax.ShapeDtypeStruct((N_TOK, D_MODEL), jnp.float32),
    mesh=plsc.VectorSubcoreMesh(core_axis_name="c", subcore_axis_name="s"),
    compiler_params=pltpu.CompilerParams(needs_layout_passes=False),
    scratch_types=[
        pltpu.VMEM((TW,), jnp.float32),
        pltpu.VMEM((2, D_MODEL), jnp.float32),
        pltpu.VMEM((C, D_MODEL), jnp.float32),
        pltpu.VMEM((C, D_MODEL), jnp.float32),
        pltpu.VMEM((C, D_MODEL), jnp.float32),
        pltpu.SemaphoreType.DMA,
        pltpu.SemaphoreType.DMA,
        pltpu.SemaphoreType.DMA,
    ],
)
def _sc_emit(xf_hbm, proj_hbm, out_hbm, xf_v, proj_v, b0, b1, b2,
             ws0, ws1, ws2):
    cid = lax.axis_index("c")
    sid = lax.axis_index("s")
    wid = sid * NC + cid
    base = wid * TW
    pltpu.sync_copy(
        xf_hbm.at[wid // (L_DIM // TW), pl.ds((wid % (L_DIM // TW)) * TW, TW)],
        xf_v)
    pltpu.sync_copy(proj_hbm, proj_v)

    bufs = (b0, b1, b2)
    wsems = (ws0, ws1, ws2)
    lanes = lax.iota(jnp.int32, 16)

    def build(ci, p):
        xv = xf_v[pl.ds(ci * C, 16)]
        ws = []
        for i in range(C):
            si = jnp.sum(jnp.where(lanes == i, xv, 0.0))
            ws.append(jnp.full((16,), si, jnp.float32))

        def col(j, carry):
            o = j * 16
            p0 = proj_v[0, pl.ds(o, 16)]
            p1 = proj_v[1, pl.ds(o, 16)]
            d = p1 - p0
            for i in range(C):
                bufs[p][i, pl.ds(o, 16)] = p0 + ws[i] * d
            return carry

        lax.fori_loop(0, D_MODEL // 16, col, 0, unroll=8)

    def start_write(ci, p):
        pltpu.async_copy(bufs[p], out_hbm.at[pl.ds(base + ci * C, C)], wsems[p])

    def wait_write(ci, p):
        pltpu.make_async_copy(
            bufs[p], out_hbm.at[pl.ds(base + ci * C, C)], wsems[p]
        ).wait()

    build(0, 0)
    start_write(0, 0)
    build(1, 1)
    start_write(1, 1)

    def body(g, carry):
        for bq in range(NBUF):
            ci = 2 + g * NBUF + bq
            p = (2 + bq) % NBUF

            @pl.when(ci >= NBUF)
            def _():
                wait_write(ci - NBUF, p)

            build(ci, p)
            start_write(ci, p)
        return carry

    lax.fori_loop(0, (NCHUNK - 2) // NBUF, body, 0)
    wait_write(NCHUNK - 3, (NCHUNK - 3) % NBUF)
    wait_write(NCHUNK - 2, (NCHUNK - 2) % NBUF)
    wait_write(NCHUNK - 1, (NCHUNK - 1) % NBUF)


def kernel(x, emb_table, W, b):
    proj, xf = _tc_prep(x.astype(jnp.int32), emb_table, W, b)
    out = _sc_emit(xf, proj)
    return out.reshape(B_DIM, L_DIM, D_MODEL)
